# Initial kernel scaffold; baseline (speedup 1.0000x reference)
import jax
import jax.numpy as jnp
from jax import lax
from jax.experimental import pallas as pl
from jax.experimental.pallas import tpu as pltpu
from jax.experimental.pallas import tpu_sc as plsc

N_NODES = 10000
N_EDGES = 160000
VOCAB = 1000
D_IN = 1024
D_H = 256
HALF = 128
NC = 2
NS = 16
LANES = 16
K = 80
EPT = N_EDGES // NS
KCH = EPT // K
NBLK = N_NODES // K
BPT = (NBLK + NS - 1) // NS
ZROWS = 125
DEG_PAD = NS * 640

f32 = jnp.float32
i32 = jnp.int32


def _proj1_body(emb_ref, ws_ref, wn_ref, ps0_ref, ps1_ref, pn0_ref, pn1_ref):
    e = emb_ref[...]
    ps = jnp.dot(e, ws_ref[...], preferred_element_type=f32)
    pn = jnp.dot(e, wn_ref[...], preferred_element_type=f32)
    ps0_ref[...] = ps[:, :HALF]
    ps1_ref[...] = ps[:, HALF:]
    pn0_ref[...] = pn[:, :HALF]
    pn1_ref[...] = pn[:, HALF:]


def _project_l1(emb, W_self1, W_neigh1):
    out = jax.ShapeDtypeStruct((VOCAB, HALF), f32)
    return pl.pallas_call(_proj1_body, out_shape=(out,) * 4)(
        emb, W_self1, W_neigh1)


def _proj2_body(h1a_ref, h1b_ref, ws_ref, wn_ref, s0_ref, s1_ref, g0_ref, g1_ref):
    a = h1a_ref[...]
    b = h1b_ref[...]
    ws = ws_ref[...]
    wn = wn_ref[...]
    s = (jnp.dot(a, ws[:HALF, :], preferred_element_type=f32)
         + jnp.dot(b, ws[HALF:, :], preferred_element_type=f32))
    g = (jnp.dot(a, wn[:HALF, :], preferred_element_type=f32)
         + jnp.dot(b, wn[HALF:, :], preferred_element_type=f32))
    s0_ref[...] = s[:, :HALF]
    s1_ref[...] = s[:, HALF:]
    g0_ref[...] = g[:, :HALF]
    g1_ref[...] = g[:, HALF:]


def _project_l2(h1a, h1b, W_self2, W_neigh2):
    R = 1000
    bs_in = pl.BlockSpec((R, HALF), lambda i: (i, 0))
    bs_w = pl.BlockSpec((D_H, D_H), lambda i: (0, 0))
    bs_out = pl.BlockSpec((R, HALF), lambda i: (i, 0))
    out = jax.ShapeDtypeStruct((N_NODES, HALF), f32)
    return pl.pallas_call(
        _proj2_body,
        grid=(N_NODES // R,),
        in_specs=[bs_in, bs_in, bs_w, bs_w],
        out_specs=(bs_out,) * 4,
        out_shape=(out,) * 4,
    )(h1a, h1b, W_self2, W_neigh2)


def _zero_vmem_2d(ref, nrows):
    def zrow(i, carry):
        for jj in range(HALF // LANES):
            ref[i, pl.ds(jj * LANES, LANES)] = jnp.zeros((LANES,), f32)
        return carry
    lax.fori_loop(0, nrows, zrow, None)


def _fill_vmem_1d(ref, n, value):
    def fill(i, carry):
        ref[pl.ds(i * LANES, LANES)] = jnp.full((LANES,), value, f32)
        return carry
    lax.fori_loop(0, n // LANES, fill, None)


def _sc1_body(node_hbm, src_hbm, dst_hbm, ps0, ps1, pn0, pn1, b1_hbm,
              h1a, h1b, rdeg_out,
              acc, deg, isrc, idst, insrc, rows, rows2, degblk, ones,
              b1buf, zbuf, zdeg, nidx, rdbuf, sem):
    c = lax.axis_index("c")
    s = lax.axis_index("s")

    _zero_vmem_2d(zbuf, ZROWS)
    _fill_vmem_1d(zdeg, 640, 0.0)
    _fill_vmem_1d(ones, K, 1.0)
    for kk in range(5):
        pltpu.sync_copy(zbuf, acc.at[pl.ds(s * 625 + kk * ZROWS, ZROWS)])
    pltpu.sync_copy(zdeg, deg.at[pl.ds(s * 640, 640)])
    plsc.subcore_barrier()

    pltpu.sync_copy(src_hbm.at[pl.ds(s * KCH, KCH)], isrc)
    pltpu.sync_copy(dst_hbm.at[pl.ds(s * KCH, KCH)], idst)
    pltpu.async_copy(node_hbm.at[isrc], insrc, sem).wait()

    def agg(pn_ref):
        def body(k, carry):
            pltpu.async_copy(pn_ref.at[insrc.at[k]], rows, sem).wait()
            pltpu.sync_copy(rows, acc.at[idst.at[k]], add=True)
            pltpu.sync_copy(ones, deg.at[idst.at[k]], add=True)
            return carry
        lax.fori_loop(0, KCH, body, None)

    @pl.when(c == 0)
    def _():
        agg(pn0)

    @pl.when(c == 1)
    def _():
        agg(pn1)

    plsc.subcore_barrier()

    pltpu.sync_copy(b1_hbm.at[pl.ds(c * HALF, HALF)], b1buf)

    def combine(ps_ref, hout_ref, do_rdeg):
        def blk(j, carry):
            b = s * BPT + j

            @pl.when(b < NBLK)
            def _():
                base = b * K
                pltpu.sync_copy(node_hbm.at[pl.ds(base, K)], nidx)
                pltpu.async_copy(ps_ref.at[nidx], rows, sem).wait()
                pltpu.sync_copy(acc.at[pl.ds(base, K)], rows2)
                pltpu.sync_copy(deg.at[pl.ds(base, K)], degblk)

                def rowfn(i, carry2):
                    dv = plsc.load_gather(degblk, [jnp.full((LANES,), i, i32)])
                    rd = 1.0 / jnp.maximum(dv, 1.0)
                    for jj in range(HALF // LANES):
                        sl = pl.ds(jj * LANES, LANES)
                        v = rows[i, sl] + rows2[i, sl] * rd + b1buf[sl]
                        rows[i, sl] = jnp.maximum(v, 0.0)
                    return carry2
                lax.fori_loop(0, K, rowfn, None)
                pltpu.sync_copy(rows, hout_ref.at[pl.ds(base, K)])
                if do_rdeg:
                    def rv(i, carry2):
                        sl = pl.ds(i * LANES, LANES)
                        rdbuf[sl] = 1.0 / jnp.maximum(degblk[sl], 1.0)
                        return carry2
                    lax.fori_loop(0, K // LANES, rv, None)
                    pltpu.sync_copy(rdbuf, rdeg_out.at[pl.ds(base, K)])
            return carry
        lax.fori_loop(0, BPT, blk, None)

    @pl.when(c == 0)
    def _():
        combine(ps0, h1a, True)

    @pl.when(c == 1)
    def _():
        combine(ps1, h1b, False)


def _sc_layer1(node, src2d, dst2d, ps0, ps1, pn0, pn1, b1):
    mesh = plsc.VectorSubcoreMesh(core_axis_name="c", subcore_axis_name="s",
                                  num_cores=NC, num_subcores=NS)
    kern = pl.kernel(
        _sc1_body,
        out_type=[jax.ShapeDtypeStruct((N_NODES, HALF), f32),
                  jax.ShapeDtypeStruct((N_NODES, HALF), f32),
                  jax.ShapeDtypeStruct((N_NODES,), f32)],
        mesh=mesh,
        scratch_types=[
            pltpu.VMEM_SHARED((N_NODES, HALF), f32),
            pltpu.VMEM_SHARED((DEG_PAD,), f32),
            pltpu.VMEM((KCH, K), i32),
            pltpu.VMEM((KCH, K), i32),
            pltpu.VMEM((KCH, K), i32),
            pltpu.VMEM((K, HALF), f32),
            pltpu.VMEM((K, HALF), f32),
            pltpu.VMEM((K,), f32),
            pltpu.VMEM((K,), f32),
            pltpu.VMEM((HALF,), f32),
            pltpu.VMEM((ZROWS, HALF), f32),
            pltpu.VMEM((640,), f32),
            pltpu.VMEM((K,), i32),
            pltpu.VMEM((K,), f32),
            pltpu.SemaphoreType.DMA,
        ],
    )
    return kern(node, src2d, dst2d, ps0, ps1, pn0, pn1, b1)


def _sc2_body(src_hbm, dst_hbm, s2a, s2b, g2a, g2b, rdeg_hbm, b2_hbm,
              h2a, h2b,
              acc, isrc, idst, rows, rows2, degblk, b2buf, zbuf, sem):
    c = lax.axis_index("c")
    s = lax.axis_index("s")

    _zero_vmem_2d(zbuf, ZROWS)
    for kk in range(5):
        pltpu.sync_copy(zbuf, acc.at[pl.ds(s * 625 + kk * ZROWS, ZROWS)])
    plsc.subcore_barrier()

    pltpu.sync_copy(src_hbm.at[pl.ds(s * KCH, KCH)], isrc)
    pltpu.sync_copy(dst_hbm.at[pl.ds(s * KCH, KCH)], idst)

    def agg(g_ref):
        def body(k, carry):
            pltpu.async_copy(g_ref.at[isrc.at[k]], rows, sem).wait()
            pltpu.sync_copy(rows, acc.at[idst.at[k]], add=True)
            return carry
        lax.fori_loop(0, KCH, body, None)

    @pl.when(c == 0)
    def _():
        agg(g2a)

    @pl.when(c == 1)
    def _():
        agg(g2b)

    plsc.subcore_barrier()

    pltpu.sync_copy(b2_hbm.at[pl.ds(c * HALF, HALF)], b2buf)

    def combine(s_ref, hout_ref):
        def blk(j, carry):
            b = s * BPT + j

            @pl.when(b < NBLK)
            def _():
                base = b * K
                pltpu.sync_copy(s_ref.at[pl.ds(base, K)], rows)
                pltpu.sync_copy(acc.at[pl.ds(base, K)], rows2)
                pltpu.sync_copy(rdeg_hbm.at[pl.ds(base, K)], degblk)

                def rowfn(i, carry2):
                    rd = plsc.load_gather(degblk, [jnp.full((LANES,), i, i32)])
                    for jj in range(HALF // LANES):
                        sl = pl.ds(jj * LANES, LANES)
                        rows[i, sl] = rows[i, sl] + rows2[i, sl] * rd + b2buf[sl]
                    return carry2
                lax.fori_loop(0, K, rowfn, None)
                pltpu.sync_copy(rows, hout_ref.at[pl.ds(base, K)])
            return carry
        lax.fori_loop(0, BPT, blk, None)

    @pl.when(c == 0)
    def _():
        combine(s2a, h2a)

    @pl.when(c == 1)
    def _():
        combine(s2b, h2b)


def _sc_layer2(src2d, dst2d, s2a, s2b, g2a, g2b, rdeg, b2):
    mesh = plsc.VectorSubcoreMesh(core_axis_name="c", subcore_axis_name="s",
                                  num_cores=NC, num_subcores=NS)
    kern = pl.kernel(
        _sc2_body,
        out_type=[jax.ShapeDtypeStruct((N_NODES, HALF), f32),
                  jax.ShapeDtypeStruct((N_NODES, HALF), f32)],
        mesh=mesh,
        scratch_types=[
            pltpu.VMEM_SHARED((N_NODES, HALF), f32),
            pltpu.VMEM((KCH, K), i32),
            pltpu.VMEM((KCH, K), i32),
            pltpu.VMEM((K, HALF), f32),
            pltpu.VMEM((K, HALF), f32),
            pltpu.VMEM((K,), f32),
            pltpu.VMEM((HALF,), f32),
            pltpu.VMEM((ZROWS, HALF), f32),
            pltpu.SemaphoreType.DMA,
        ],
    )
    return kern(src2d, dst2d, s2a, s2b, g2a, g2b, rdeg, b2)


def kernel(node, edge_index, emb, W_self1, W_neigh1, b1, W_self2, W_neigh2, b2):
    src2d = edge_index[0].reshape(N_EDGES // K, K)
    dst2d = edge_index[1].reshape(N_EDGES // K, K)
    ps0, ps1, pn0, pn1 = _project_l1(emb, W_self1, W_neigh1)
    h1a, h1b, rdeg = _sc_layer1(node, src2d, dst2d, ps0, ps1, pn0, pn1, b1)
    s2a, s2b, g2a, g2b = _project_l2(h1a, h1b, W_self2, W_neigh2)
    h2a, h2b = _sc_layer2(src2d, dst2d, s2a, s2b, g2a, g2b, rdeg, b2)
    return jnp.concatenate([h2a, h2b], axis=1)


# speedup vs baseline: 6.5471x; 6.5471x over previous
import jax
import jax.numpy as jnp
from jax import lax
from jax.experimental import pallas as pl
from jax.experimental.pallas import tpu as pltpu
from jax.experimental.pallas import tpu_sc as plsc

N_NODES = 10000
N_EDGES = 160000
VOCAB = 1000
D_IN = 1024
D_H = 256
HALF = 128
NC = 2
NS = 16
LANES = 16
K = 80
EPT = N_EDGES // NS
KCH = EPT // K
NBLK = N_NODES // K
BPT = (NBLK + NS - 1) // NS
ZROWS = 125
DEG_PAD = NS * 640

f32 = jnp.float32
i32 = jnp.int32


def _proj1_body(emb_ref, ws_ref, wn_ref, ps0_ref, ps1_ref, pn0_ref, pn1_ref):
    e = emb_ref[...]
    ps = jnp.dot(e, ws_ref[...], preferred_element_type=f32)
    pn = jnp.dot(e, wn_ref[...], preferred_element_type=f32)
    ps0_ref[...] = ps[:, :HALF]
    ps1_ref[...] = ps[:, HALF:]
    pn0_ref[...] = pn[:, :HALF]
    pn1_ref[...] = pn[:, HALF:]


def _project_l1(emb, W_self1, W_neigh1):
    out = jax.ShapeDtypeStruct((VOCAB, HALF), f32)
    return pl.pallas_call(_proj1_body, out_shape=(out,) * 4)(
        emb, W_self1, W_neigh1)


def _proj2_body(h1a_ref, h1b_ref, ws_ref, wn_ref, s0_ref, s1_ref, g0_ref, g1_ref):
    a = h1a_ref[...]
    b = h1b_ref[...]
    ws = ws_ref[...]
    wn = wn_ref[...]
    s = (jnp.dot(a, ws[:HALF, :], preferred_element_type=f32)
         + jnp.dot(b, ws[HALF:, :], preferred_element_type=f32))
    g = (jnp.dot(a, wn[:HALF, :], preferred_element_type=f32)
         + jnp.dot(b, wn[HALF:, :], preferred_element_type=f32))
    s0_ref[...] = s[:, :HALF]
    s1_ref[...] = s[:, HALF:]
    g0_ref[...] = g[:, :HALF]
    g1_ref[...] = g[:, HALF:]


def _project_l2(h1a, h1b, W_self2, W_neigh2):
    R = 1000
    bs_in = pl.BlockSpec((R, HALF), lambda i: (i, 0))
    bs_w = pl.BlockSpec((D_H, D_H), lambda i: (0, 0))
    bs_out = pl.BlockSpec((R, HALF), lambda i: (i, 0))
    out = jax.ShapeDtypeStruct((N_NODES, HALF), f32)
    return pl.pallas_call(
        _proj2_body,
        grid=(N_NODES // R,),
        in_specs=[bs_in, bs_in, bs_w, bs_w],
        out_specs=(bs_out,) * 4,
        out_shape=(out,) * 4,
    )(h1a, h1b, W_self2, W_neigh2)


def _zero_vmem_2d(ref, nrows):
    def zrow(i, carry):
        for jj in range(HALF // LANES):
            ref[i, pl.ds(jj * LANES, LANES)] = jnp.zeros((LANES,), f32)
        return carry
    lax.fori_loop(0, nrows, zrow, None)


def _fill_vmem_1d(ref, n, value):
    def fill(i, carry):
        ref[pl.ds(i * LANES, LANES)] = jnp.full((LANES,), value, f32)
        return carry
    lax.fori_loop(0, n // LANES, fill, None)


def _sc1_body(node_hbm, src_hbm, dst_hbm, ps0, ps1, pn0, pn1, b1_hbm,
              h1a, h1b, rdeg_out,
              acc, deg, isrc, idst, nsbuf, rows, rows2, degblk,
              b1buf, nidx, rdbuf, sem):
    c = lax.axis_index("c")
    s = lax.axis_index("s")

    _zero_vmem_2d(rows, K)
    _fill_vmem_1d(rdbuf, K, 0.0)
    def zblk(q, carry):
        b = s * BPT + q

        @pl.when(b < NBLK)
        def _():
            pltpu.sync_copy(rows, acc.at[pl.ds(b * K, K)])
        return carry
    lax.fori_loop(0, BPT, zblk, None)
    for q in range(8):
        pltpu.sync_copy(rdbuf, deg.at[pl.ds(s * 640 + q * K, K)])
    _fill_vmem_1d(rdbuf, K, 1.0)
    plsc.subcore_barrier()

    pltpu.sync_copy(src_hbm.at[pl.ds(s * EPT, EPT)], isrc)
    pltpu.sync_copy(dst_hbm.at[s], idst)

    def agg(pn_ref):
        def body(k, carry):
            pltpu.async_copy(node_hbm.at[isrc.at[pl.ds(k * K, K)]], nsbuf,
                             sem).wait()
            pltpu.async_copy(pn_ref.at[nsbuf], rows, sem).wait()
            pltpu.sync_copy(rows, acc.at[idst.at[k]], add=True)
            pltpu.sync_copy(rdbuf, deg.at[idst.at[k]], add=True)
            return carry
        lax.fori_loop(0, KCH, body, None)

    @pl.when(c == 0)
    def _():
        agg(pn0)

    @pl.when(c == 1)
    def _():
        agg(pn1)

    plsc.subcore_barrier()

    pltpu.sync_copy(b1_hbm.at[pl.ds(c * HALF, HALF)], b1buf)

    def combine(ps_ref, hout_ref, do_rdeg):
        def blk(j, carry):
            b = s * BPT + j

            @pl.when(b < NBLK)
            def _():
                base = b * K
                pltpu.sync_copy(node_hbm.at[pl.ds(base, K)], nidx)
                pltpu.async_copy(ps_ref.at[nidx], rows, sem).wait()
                pltpu.sync_copy(acc.at[pl.ds(base, K)], rows2)
                pltpu.sync_copy(deg.at[pl.ds(base, K)], degblk)

                def tfn(t, carry2):
                    dvec = degblk[pl.ds(t * LANES, LANES)]
                    rdvec = 1.0 / jnp.maximum(dvec, 1.0)
                    for l in range(LANES):
                        i = t * LANES + l
                        rd = lax.broadcast_in_dim(
                            lax.slice(rdvec, (l,), (l + 1,)), (LANES,), (0,))
                        for jj in range(HALF // LANES):
                            sl = pl.ds(jj * LANES, LANES)
                            v = rows[i, sl] + rows2[i, sl] * rd + b1buf[sl]
                            rows[i, sl] = jnp.maximum(v, 0.0)
                    return carry2
                lax.fori_loop(0, K // LANES, tfn, None)
                pltpu.sync_copy(rows, hout_ref.at[pl.ds(base, K)])
                if do_rdeg:
                    def rv(i, carry2):
                        sl = pl.ds(i * LANES, LANES)
                        rdbuf[sl] = 1.0 / jnp.maximum(degblk[sl], 1.0)
                        return carry2
                    lax.fori_loop(0, K // LANES, rv, None)
                    pltpu.sync_copy(rdbuf, rdeg_out.at[pl.ds(base, K)])
            return carry
        lax.fori_loop(0, BPT, blk, None)

    @pl.when(c == 0)
    def _():
        combine(ps0, h1a, True)

    @pl.when(c == 1)
    def _():
        combine(ps1, h1b, False)


def _sc_layer1(node, srcf, dst3d, ps0, ps1, pn0, pn1, b1):
    mesh = plsc.VectorSubcoreMesh(core_axis_name="c", subcore_axis_name="s",
                                  num_cores=NC, num_subcores=NS)
    kern = pl.kernel(
        _sc1_body,
        out_type=[jax.ShapeDtypeStruct((N_NODES, HALF), f32),
                  jax.ShapeDtypeStruct((N_NODES, HALF), f32),
                  jax.ShapeDtypeStruct((N_NODES,), f32)],
        mesh=mesh,
        scratch_types=[
            pltpu.VMEM_SHARED((N_NODES, HALF), f32),
            pltpu.VMEM_SHARED((DEG_PAD,), f32),
            pltpu.VMEM((EPT,), i32),
            pltpu.VMEM((KCH, K), i32),
            pltpu.VMEM((K,), i32),
            pltpu.VMEM((K, HALF), f32),
            pltpu.VMEM((K, HALF), f32),
            pltpu.VMEM((K,), f32),
            pltpu.VMEM((HALF,), f32),
            pltpu.VMEM((K,), i32),
            pltpu.VMEM((K,), f32),
            pltpu.SemaphoreType.DMA,
        ],
    )
    return kern(node, srcf, dst3d, ps0, ps1, pn0, pn1, b1)


def _sc2_body(src_hbm, dst_hbm, s2a, s2b, g2a, g2b, rdeg_hbm, b2_hbm,
              h2a, h2b,
              acc, isrc, idst, rows, rows2, degblk, b2buf, sem):
    c = lax.axis_index("c")
    s = lax.axis_index("s")

    _zero_vmem_2d(rows, K)
    def zblk(q, carry):
        b = s * BPT + q

        @pl.when(b < NBLK)
        def _():
            pltpu.sync_copy(rows, acc.at[pl.ds(b * K, K)])
        return carry
    lax.fori_loop(0, BPT, zblk, None)
    plsc.subcore_barrier()

    pltpu.sync_copy(src_hbm.at[pl.ds(s * EPT, EPT)], isrc)
    pltpu.sync_copy(dst_hbm.at[s], idst)

    def agg(g_ref):
        def body(k, carry):
            pltpu.async_copy(g_ref.at[isrc.at[pl.ds(k * K, K)]], rows,
                             sem).wait()
            pltpu.sync_copy(rows, acc.at[idst.at[k]], add=True)
            return carry
        lax.fori_loop(0, KCH, body, None)

    @pl.when(c == 0)
    def _():
        agg(g2a)

    @pl.when(c == 1)
    def _():
        agg(g2b)

    plsc.subcore_barrier()

    pltpu.sync_copy(b2_hbm.at[pl.ds(c * HALF, HALF)], b2buf)

    def combine(s_ref, hout_ref):
        def blk(j, carry):
            b = s * BPT + j

            @pl.when(b < NBLK)
            def _():
                base = b * K
                pltpu.sync_copy(s_ref.at[pl.ds(base, K)], rows)
                pltpu.sync_copy(acc.at[pl.ds(base, K)], rows2)
                pltpu.sync_copy(rdeg_hbm.at[pl.ds(base, K)], degblk)

                def tfn(t, carry2):
                    rdvec = degblk[pl.ds(t * LANES, LANES)]
                    for l in range(LANES):
                        i = t * LANES + l
                        rd = lax.broadcast_in_dim(
                            lax.slice(rdvec, (l,), (l + 1,)), (LANES,), (0,))
                        for jj in range(HALF // LANES):
                            sl = pl.ds(jj * LANES, LANES)
                            rows[i, sl] = (rows[i, sl] + rows2[i, sl] * rd
                                           + b2buf[sl])
                    return carry2
                lax.fori_loop(0, K // LANES, tfn, None)
                pltpu.sync_copy(rows, hout_ref.at[pl.ds(base, K)])
            return carry
        lax.fori_loop(0, BPT, blk, None)

    @pl.when(c == 0)
    def _():
        combine(s2a, h2a)

    @pl.when(c == 1)
    def _():
        combine(s2b, h2b)


def _sc_layer2(srcf, dst3d, s2a, s2b, g2a, g2b, rdeg, b2):
    mesh = plsc.VectorSubcoreMesh(core_axis_name="c", subcore_axis_name="s",
                                  num_cores=NC, num_subcores=NS)
    kern = pl.kernel(
        _sc2_body,
        out_type=[jax.ShapeDtypeStruct((N_NODES, HALF), f32),
                  jax.ShapeDtypeStruct((N_NODES, HALF), f32)],
        mesh=mesh,
        scratch_types=[
            pltpu.VMEM_SHARED((N_NODES, HALF), f32),
            pltpu.VMEM((EPT,), i32),
            pltpu.VMEM((KCH, K), i32),
            pltpu.VMEM((K, HALF), f32),
            pltpu.VMEM((K, HALF), f32),
            pltpu.VMEM((K,), f32),
            pltpu.VMEM((HALF,), f32),
            pltpu.SemaphoreType.DMA,
        ],
    )
    return kern(srcf, dst3d, s2a, s2b, g2a, g2b, rdeg, b2)


def kernel(node, edge_index, emb, W_self1, W_neigh1, b1, W_self2, W_neigh2, b2):
    srcf = edge_index[0]
    dst3d = edge_index[1].reshape(NS, KCH, K)
    ps0, ps1, pn0, pn1 = _project_l1(emb, W_self1, W_neigh1)
    h1a, h1b, rdeg = _sc_layer1(node, srcf, dst3d, ps0, ps1, pn0, pn1, b1)
    s2a, s2b, g2a, g2b = _project_l2(h1a, h1b, W_self2, W_neigh2)
    h2a, h2b = _sc_layer2(srcf, dst3d, s2a, s2b, g2a, g2b, rdeg, b2)
    return jnp.concatenate([h2a, h2b], axis=1)


# speedup vs baseline: 9.1197x; 1.3929x over previous
import jax
import jax.numpy as jnp
from jax import lax
from jax.experimental import pallas as pl
from jax.experimental.pallas import tpu as pltpu
from jax.experimental.pallas import tpu_sc as plsc

N_NODES = 10000
N_EDGES = 160000
VOCAB = 1000
D_IN = 1024
D_H = 256
HALF = 128
NC = 2
NS = 16
LANES = 16
K = 80
EPT = N_EDGES // NS
KCH = EPT // K
NBLK = N_NODES // K
BPT = (NBLK + NS - 1) // NS
ZROWS = 125
DEG_PAD = NS * 640

f32 = jnp.float32
i32 = jnp.int32


def _proj1_body(emb_ref, ws_ref, wn_ref, ps0_ref, ps1_ref, pn0_ref, pn1_ref):
    e = emb_ref[...]
    ps = jnp.dot(e, ws_ref[...], preferred_element_type=f32)
    pn = jnp.dot(e, wn_ref[...], preferred_element_type=f32)
    ps0_ref[...] = ps[:, :HALF]
    ps1_ref[...] = ps[:, HALF:]
    pn0_ref[...] = pn[:, :HALF]
    pn1_ref[...] = pn[:, HALF:]


def _project_l1(emb, W_self1, W_neigh1):
    out = jax.ShapeDtypeStruct((VOCAB, HALF), f32)
    return pl.pallas_call(_proj1_body, out_shape=(out,) * 4)(
        emb, W_self1, W_neigh1)


def _proj2_body(h1a_ref, h1b_ref, ws_ref, wn_ref, s0_ref, s1_ref, g0_ref, g1_ref):
    a = h1a_ref[...]
    b = h1b_ref[...]
    ws = ws_ref[...]
    wn = wn_ref[...]
    s = (jnp.dot(a, ws[:HALF, :], preferred_element_type=f32)
         + jnp.dot(b, ws[HALF:, :], preferred_element_type=f32))
    g = (jnp.dot(a, wn[:HALF, :], preferred_element_type=f32)
         + jnp.dot(b, wn[HALF:, :], preferred_element_type=f32))
    s0_ref[...] = s[:, :HALF]
    s1_ref[...] = s[:, HALF:]
    g0_ref[...] = g[:, :HALF]
    g1_ref[...] = g[:, HALF:]


def _project_l2(h1a, h1b, W_self2, W_neigh2):
    R = 1000
    bs_in = pl.BlockSpec((R, HALF), lambda i: (i, 0))
    bs_w = pl.BlockSpec((D_H, D_H), lambda i: (0, 0))
    bs_out = pl.BlockSpec((R, HALF), lambda i: (i, 0))
    out = jax.ShapeDtypeStruct((N_NODES, HALF), f32)
    return pl.pallas_call(
        _proj2_body,
        grid=(N_NODES // R,),
        in_specs=[bs_in, bs_in, bs_w, bs_w],
        out_specs=(bs_out,) * 4,
        out_shape=(out,) * 4,
    )(h1a, h1b, W_self2, W_neigh2)


def _zero_vmem_2d(ref, nrows):
    def zrow(i, carry):
        for jj in range(HALF // LANES):
            ref[i, pl.ds(jj * LANES, LANES)] = jnp.zeros((LANES,), f32)
        return carry
    lax.fori_loop(0, nrows, zrow, None)


def _fill_vmem_1d(ref, n, value):
    def fill(i, carry):
        ref[pl.ds(i * LANES, LANES)] = jnp.full((LANES,), value, f32)
        return carry
    lax.fori_loop(0, n // LANES, fill, None)


def _agg_pipe(tab_ref, isrc, idst, rows, rows2, acc, sem_a, sem_b,
              deg=None, rdbuf=None):
    def gidx(k):
        return isrc.at[pl.ds(k * K, K)]

    def scat(buf, k):
        pltpu.sync_copy(buf, acc.at[idst.at[k]], add=True)
        if deg is not None:
            pltpu.sync_copy(rdbuf, deg.at[idst.at[k]], add=True)

    pltpu.async_copy(tab_ref.at[gidx(0)], rows, sem_a)

    def body(kk, carry):
        k0 = 2 * kk
        k1 = k0 + 1
        pltpu.make_async_copy(tab_ref.at[gidx(k0)], rows, sem_a).wait()
        pltpu.async_copy(tab_ref.at[gidx(k1)], rows2, sem_b)
        scat(rows, k0)
        pltpu.make_async_copy(tab_ref.at[gidx(k1)], rows2, sem_b).wait()
        pltpu.async_copy(tab_ref.at[gidx(k0 + 2)], rows, sem_a)
        scat(rows2, k1)
        return carry
    lax.fori_loop(0, (KCH - 1) // 2, body, None)
    pltpu.make_async_copy(tab_ref.at[gidx(KCH - 1)], rows, sem_a).wait()
    scat(rows, KCH - 1)


def _sc1_body(node_hbm, src_hbm, dst_hbm, ps0, ps1, pn0, pn1, b1_hbm,
              h1a, h1b, rdeg_out, ta, tb,
              acc, deg, isrc, idst, rows, rows2, degblk,
              b1buf, nidx, rdbuf, sem_a, sem_b):
    c = lax.axis_index("c")
    s = lax.axis_index("s")

    _zero_vmem_2d(rows, K)
    _fill_vmem_1d(rdbuf, K, 0.0)
    def zblk(q, carry):
        b = s * BPT + q

        @pl.when(b < NBLK)
        def _():
            pltpu.sync_copy(rows, acc.at[pl.ds(b * K, K)])
        return carry
    lax.fori_loop(0, BPT, zblk, None)
    for q in range(8):
        pltpu.sync_copy(rdbuf, deg.at[pl.ds(s * 640 + q * K, K)])
    _fill_vmem_1d(rdbuf, K, 1.0)

    def tbuild(pn_ref, t_ref):
        def blk(q, carry):
            b = s * BPT + q

            @pl.when(b < NBLK)
            def _():
                base = b * K
                pltpu.sync_copy(node_hbm.at[pl.ds(base, K)], nidx)
                pltpu.async_copy(pn_ref.at[nidx], rows2, sem_a).wait()
                pltpu.sync_copy(rows2, t_ref.at[pl.ds(base, K)])
            return carry
        lax.fori_loop(0, BPT, blk, None)

    @pl.when(c == 0)
    def _():
        tbuild(pn0, ta)

    @pl.when(c == 1)
    def _():
        tbuild(pn1, tb)

    pltpu.sync_copy(src_hbm.at[pl.ds(s * EPT, EPT)], isrc)
    pltpu.sync_copy(dst_hbm.at[s], idst)
    plsc.subcore_barrier()

    @pl.when(c == 0)
    def _():
        _agg_pipe(ta, isrc, idst, rows, rows2, acc, sem_a, sem_b, deg, rdbuf)

    @pl.when(c == 1)
    def _():
        _agg_pipe(tb, isrc, idst, rows, rows2, acc, sem_a, sem_b, deg, rdbuf)

    plsc.subcore_barrier()

    pltpu.sync_copy(b1_hbm.at[pl.ds(c * HALF, HALF)], b1buf)

    def combine(ps_ref, hout_ref, do_rdeg):
        def blk(j, carry):
            b = s * BPT + j

            @pl.when(b < NBLK)
            def _():
                base = b * K
                pltpu.sync_copy(node_hbm.at[pl.ds(base, K)], nidx)
                pltpu.async_copy(ps_ref.at[nidx], rows, sem_a).wait()
                pltpu.sync_copy(acc.at[pl.ds(base, K)], rows2)
                pltpu.sync_copy(deg.at[pl.ds(base, K)], degblk)

                def tfn(t, carry2):
                    dvec = degblk[pl.ds(t * LANES, LANES)]
                    rdvec = 1.0 / jnp.maximum(dvec, 1.0)
                    for l in range(LANES):
                        i = t * LANES + l
                        rd = lax.broadcast_in_dim(
                            lax.slice(rdvec, (l,), (l + 1,)), (LANES,), (0,))
                        for jj in range(HALF // LANES):
                            sl = pl.ds(jj * LANES, LANES)
                            v = rows[i, sl] + rows2[i, sl] * rd + b1buf[sl]
                            rows[i, sl] = jnp.maximum(v, 0.0)
                    return carry2
                lax.fori_loop(0, K // LANES, tfn, None)
                pltpu.sync_copy(rows, hout_ref.at[pl.ds(base, K)])
                if do_rdeg:
                    def rv(i, carry2):
                        sl = pl.ds(i * LANES, LANES)
                        rdbuf[sl] = 1.0 / jnp.maximum(degblk[sl], 1.0)
                        return carry2
                    lax.fori_loop(0, K // LANES, rv, None)
                    pltpu.sync_copy(rdbuf, rdeg_out.at[pl.ds(base, K)])
            return carry
        lax.fori_loop(0, BPT, blk, None)

    @pl.when(c == 0)
    def _():
        combine(ps0, h1a, True)

    @pl.when(c == 1)
    def _():
        combine(ps1, h1b, False)


def _sc_layer1(node, srcf, dst3d, ps0, ps1, pn0, pn1, b1):
    mesh = plsc.VectorSubcoreMesh(core_axis_name="c", subcore_axis_name="s",
                                  num_cores=NC, num_subcores=NS)
    kern = pl.kernel(
        _sc1_body,
        out_type=[jax.ShapeDtypeStruct((N_NODES, HALF), f32),
                  jax.ShapeDtypeStruct((N_NODES, HALF), f32),
                  jax.ShapeDtypeStruct((N_NODES,), f32),
                  jax.ShapeDtypeStruct((N_NODES, HALF), f32),
                  jax.ShapeDtypeStruct((N_NODES, HALF), f32)],
        mesh=mesh,
        scratch_types=[
            pltpu.VMEM_SHARED((N_NODES, HALF), f32),
            pltpu.VMEM_SHARED((DEG_PAD,), f32),
            pltpu.VMEM((EPT,), i32),
            pltpu.VMEM((KCH, K), i32),
            pltpu.VMEM((K, HALF), f32),
            pltpu.VMEM((K, HALF), f32),
            pltpu.VMEM((K,), f32),
            pltpu.VMEM((HALF,), f32),
            pltpu.VMEM((K,), i32),
            pltpu.VMEM((K,), f32),
            pltpu.SemaphoreType.DMA,
            pltpu.SemaphoreType.DMA,
        ],
    )
    h1a, h1b, rdeg, _ta, _tb = kern(node, srcf, dst3d, ps0, ps1, pn0, pn1, b1)
    return h1a, h1b, rdeg


def _sc2_body(src_hbm, dst_hbm, s2a, s2b, g2a, g2b, rdeg_hbm, b2_hbm,
              h2a, h2b,
              acc, isrc, idst, rows, rows2, degblk, b2buf, sem_a, sem_b):
    c = lax.axis_index("c")
    s = lax.axis_index("s")

    _zero_vmem_2d(rows, K)
    def zblk(q, carry):
        b = s * BPT + q

        @pl.when(b < NBLK)
        def _():
            pltpu.sync_copy(rows, acc.at[pl.ds(b * K, K)])
        return carry
    lax.fori_loop(0, BPT, zblk, None)
    plsc.subcore_barrier()

    pltpu.sync_copy(src_hbm.at[pl.ds(s * EPT, EPT)], isrc)
    pltpu.sync_copy(dst_hbm.at[s], idst)

    @pl.when(c == 0)
    def _():
        _agg_pipe(g2a, isrc, idst, rows, rows2, acc, sem_a, sem_b)

    @pl.when(c == 1)
    def _():
        _agg_pipe(g2b, isrc, idst, rows, rows2, acc, sem_a, sem_b)

    plsc.subcore_barrier()

    pltpu.sync_copy(b2_hbm.at[pl.ds(c * HALF, HALF)], b2buf)

    def combine(s_ref, hout_ref):
        def blk(j, carry):
            b = s * BPT + j

            @pl.when(b < NBLK)
            def _():
                base = b * K
                pltpu.sync_copy(s_ref.at[pl.ds(base, K)], rows)
                pltpu.sync_copy(acc.at[pl.ds(base, K)], rows2)
                pltpu.sync_copy(rdeg_hbm.at[pl.ds(base, K)], degblk)

                def tfn(t, carry2):
                    rdvec = degblk[pl.ds(t * LANES, LANES)]
                    for l in range(LANES):
                        i = t * LANES + l
                        rd = lax.broadcast_in_dim(
                            lax.slice(rdvec, (l,), (l + 1,)), (LANES,), (0,))
                        for jj in range(HALF // LANES):
                            sl = pl.ds(jj * LANES, LANES)
                            rows[i, sl] = (rows[i, sl] + rows2[i, sl] * rd
                                           + b2buf[sl])
                    return carry2
                lax.fori_loop(0, K // LANES, tfn, None)
                pltpu.sync_copy(rows, hout_ref.at[pl.ds(base, K)])
            return carry
        lax.fori_loop(0, BPT, blk, None)

    @pl.when(c == 0)
    def _():
        combine(s2a, h2a)

    @pl.when(c == 1)
    def _():
        combine(s2b, h2b)


def _sc_layer2(srcf, dst3d, s2a, s2b, g2a, g2b, rdeg, b2):
    mesh = plsc.VectorSubcoreMesh(core_axis_name="c", subcore_axis_name="s",
                                  num_cores=NC, num_subcores=NS)
    kern = pl.kernel(
        _sc2_body,
        out_type=[jax.ShapeDtypeStruct((N_NODES, HALF), f32),
                  jax.ShapeDtypeStruct((N_NODES, HALF), f32)],
        mesh=mesh,
        scratch_types=[
            pltpu.VMEM_SHARED((N_NODES, HALF), f32),
            pltpu.VMEM((EPT,), i32),
            pltpu.VMEM((KCH, K), i32),
            pltpu.VMEM((K, HALF), f32),
            pltpu.VMEM((K, HALF), f32),
            pltpu.VMEM((K,), f32),
            pltpu.VMEM((HALF,), f32),
            pltpu.SemaphoreType.DMA,
            pltpu.SemaphoreType.DMA,
        ],
    )
    return kern(srcf, dst3d, s2a, s2b, g2a, g2b, rdeg, b2)


def kernel(node, edge_index, emb, W_self1, W_neigh1, b1, W_self2, W_neigh2, b2):
    srcf = edge_index[0]
    dst3d = edge_index[1].reshape(NS, KCH, K)
    ps0, ps1, pn0, pn1 = _project_l1(emb, W_self1, W_neigh1)
    h1a, h1b, rdeg = _sc_layer1(node, srcf, dst3d, ps0, ps1, pn0, pn1, b1)
    s2a, s2b, g2a, g2b = _project_l2(h1a, h1b, W_self2, W_neigh2)
    h2a, h2b = _sc_layer2(srcf, dst3d, s2a, s2b, g2a, g2b, rdeg, b2)
    return jnp.concatenate([h2a, h2b], axis=1)


# speedup vs baseline: 9.2525x; 1.0146x over previous
import jax
import jax.numpy as jnp
from jax import lax
from jax.experimental import pallas as pl
from jax.experimental.pallas import tpu as pltpu
from jax.experimental.pallas import tpu_sc as plsc

N_NODES = 10000
N_EDGES = 160000
VOCAB = 1000
D_IN = 1024
D_H = 256
HALF = 128
NC = 2
NS = 16
LANES = 16
K = 80
EPT = N_EDGES // NS
KCH = EPT // K
NBLK = N_NODES // K
BPT = (NBLK + NS - 1) // NS
ZROWS = 125
DEG_PAD = NS * 640

f32 = jnp.float32
i32 = jnp.int32


def _proj1_body(emb_ref, ws_ref, wn_ref, ps0_ref, ps1_ref, pn0_ref, pn1_ref):
    e = emb_ref[...]
    ps = jnp.dot(e, ws_ref[...], preferred_element_type=f32)
    pn = jnp.dot(e, wn_ref[...], preferred_element_type=f32)
    ps0_ref[...] = ps[:, :HALF]
    ps1_ref[...] = ps[:, HALF:]
    pn0_ref[...] = pn[:, :HALF]
    pn1_ref[...] = pn[:, HALF:]


def _project_l1(emb, W_self1, W_neigh1):
    out = jax.ShapeDtypeStruct((VOCAB, HALF), f32)
    return pl.pallas_call(_proj1_body, out_shape=(out,) * 4)(
        emb, W_self1, W_neigh1)


def _proj2_body(h1a_ref, h1b_ref, ws_ref, wn_ref, s0_ref, s1_ref, g0_ref, g1_ref):
    a = h1a_ref[...]
    b = h1b_ref[...]
    ws = ws_ref[...]
    wn = wn_ref[...]
    s = (jnp.dot(a, ws[:HALF, :], preferred_element_type=f32)
         + jnp.dot(b, ws[HALF:, :], preferred_element_type=f32))
    g = (jnp.dot(a, wn[:HALF, :], preferred_element_type=f32)
         + jnp.dot(b, wn[HALF:, :], preferred_element_type=f32))
    s0_ref[...] = s[:, :HALF]
    s1_ref[...] = s[:, HALF:]
    g0_ref[...] = g[:, :HALF]
    g1_ref[...] = g[:, HALF:]


def _project_l2(h1a, h1b, W_self2, W_neigh2):
    R = 1000
    bs_in = pl.BlockSpec((R, HALF), lambda i: (i, 0))
    bs_w = pl.BlockSpec((D_H, D_H), lambda i: (0, 0))
    bs_out = pl.BlockSpec((R, HALF), lambda i: (i, 0))
    out = jax.ShapeDtypeStruct((N_NODES, HALF), f32)
    return pl.pallas_call(
        _proj2_body,
        grid=(N_NODES // R,),
        in_specs=[bs_in, bs_in, bs_w, bs_w],
        out_specs=(bs_out,) * 4,
        out_shape=(out,) * 4,
    )(h1a, h1b, W_self2, W_neigh2)


def _zero_vmem_2d(ref, nrows):
    def zrow(i, carry):
        for jj in range(HALF // LANES):
            ref[i, pl.ds(jj * LANES, LANES)] = jnp.zeros((LANES,), f32)
        return carry
    lax.fori_loop(0, nrows, zrow, None)


def _fill_vmem_1d(ref, n, value):
    def fill(i, carry):
        ref[pl.ds(i * LANES, LANES)] = jnp.full((LANES,), value, f32)
        return carry
    lax.fori_loop(0, n // LANES, fill, None)


def _agg_pipe(tab_ref, isrc, idst, rows, rows2, acc, sem_a, sem_b,
              deg=None, rdbuf=None, sem_d=None):
    def gidx(k):
        return isrc.at[pl.ds(k * K, K)]

    def scat(buf, k):
        pltpu.sync_copy(buf, acc.at[idst.at[k]], add=True)
        if deg is not None:
            pltpu.async_copy(rdbuf, deg.at[idst.at[k]], sem_d, add=True)

    pltpu.async_copy(tab_ref.at[gidx(0)], rows, sem_a)

    def body(kk, carry):
        k0 = 2 * kk
        k1 = k0 + 1
        pltpu.make_async_copy(tab_ref.at[gidx(k0)], rows, sem_a).wait()
        pltpu.async_copy(tab_ref.at[gidx(k1)], rows2, sem_b)
        scat(rows, k0)
        pltpu.make_async_copy(tab_ref.at[gidx(k1)], rows2, sem_b).wait()
        pltpu.async_copy(tab_ref.at[gidx(k0 + 2)], rows, sem_a)
        scat(rows2, k1)
        return carry
    lax.fori_loop(0, (KCH - 1) // 2, body, None)
    pltpu.make_async_copy(tab_ref.at[gidx(KCH - 1)], rows, sem_a).wait()
    scat(rows, KCH - 1)
    if deg is not None:
        def drain(k, carry):
            pltpu.make_async_copy(rdbuf, deg.at[idst.at[0]], sem_d).wait()
            return carry
        lax.fori_loop(0, KCH, drain, None)


def _sc1_body(node_hbm, src_hbm, dst_hbm, ps0, ps1, pn0, pn1, b1_hbm,
              h1a, h1b, rdeg_out, ta, tb,
              acc, deg, isrc, idst, rows, rows2, degblk,
              b1buf, nidx, rdbuf, sem_a, sem_b, sem_d):
    c = lax.axis_index("c")
    s = lax.axis_index("s")

    _zero_vmem_2d(rows, K)
    _fill_vmem_1d(rdbuf, K, 0.0)
    def zblk(q, carry):
        b = s * BPT + q

        @pl.when(b < NBLK)
        def _():
            pltpu.sync_copy(rows, acc.at[pl.ds(b * K, K)])
        return carry
    lax.fori_loop(0, BPT, zblk, None)
    for q in range(8):
        pltpu.sync_copy(rdbuf, deg.at[pl.ds(s * 640 + q * K, K)])
    _fill_vmem_1d(rdbuf, K, 1.0)

    def tbuild(pn_ref, t_ref):
        def blk(q, carry):
            b = s * BPT + q

            @pl.when(b < NBLK)
            def _():
                base = b * K
                pltpu.sync_copy(node_hbm.at[pl.ds(base, K)], nidx)
                pltpu.async_copy(pn_ref.at[nidx], rows2, sem_a).wait()
                pltpu.sync_copy(rows2, t_ref.at[pl.ds(base, K)])
            return carry
        lax.fori_loop(0, BPT, blk, None)

    @pl.when(c == 0)
    def _():
        tbuild(pn0, ta)

    @pl.when(c == 1)
    def _():
        tbuild(pn1, tb)

    pltpu.sync_copy(src_hbm.at[pl.ds(s * EPT, EPT)], isrc)
    pltpu.sync_copy(dst_hbm.at[s], idst)
    plsc.subcore_barrier()

    @pl.when(c == 0)
    def _():
        _agg_pipe(ta, isrc, idst, rows, rows2, acc, sem_a, sem_b, deg, rdbuf,
                  sem_d)

    @pl.when(c == 1)
    def _():
        _agg_pipe(tb, isrc, idst, rows, rows2, acc, sem_a, sem_b, deg, rdbuf,
                  sem_d)

    plsc.subcore_barrier()

    pltpu.sync_copy(b1_hbm.at[pl.ds(c * HALF, HALF)], b1buf)

    def combine(ps_ref, hout_ref, do_rdeg):
        def blk(j, carry):
            b = s * BPT + j

            @pl.when(b < NBLK)
            def _():
                base = b * K
                pltpu.sync_copy(node_hbm.at[pl.ds(base, K)], nidx)
                pltpu.async_copy(ps_ref.at[nidx], rows, sem_a).wait()
                pltpu.sync_copy(acc.at[pl.ds(base, K)], rows2)
                pltpu.sync_copy(deg.at[pl.ds(base, K)], degblk)

                def tfn(t, carry2):
                    dvec = degblk[pl.ds(t * LANES, LANES)]
                    rdvec = 1.0 / jnp.maximum(dvec, 1.0)
                    for l in range(LANES):
                        i = t * LANES + l
                        rd = lax.broadcast_in_dim(
                            lax.slice(rdvec, (l,), (l + 1,)), (LANES,), (0,))
                        for jj in range(HALF // LANES):
                            sl = pl.ds(jj * LANES, LANES)
                            v = rows[i, sl] + rows2[i, sl] * rd + b1buf[sl]
                            rows[i, sl] = jnp.maximum(v, 0.0)
                    return carry2
                lax.fori_loop(0, K // LANES, tfn, None)
                pltpu.sync_copy(rows, hout_ref.at[pl.ds(base, K)])
                if do_rdeg:
                    def rv(i, carry2):
                        sl = pl.ds(i * LANES, LANES)
                        rdbuf[sl] = 1.0 / jnp.maximum(degblk[sl], 1.0)
                        return carry2
                    lax.fori_loop(0, K // LANES, rv, None)
                    pltpu.sync_copy(rdbuf, rdeg_out.at[pl.ds(base, K)])
            return carry
        lax.fori_loop(0, BPT, blk, None)

    @pl.when(c == 0)
    def _():
        combine(ps0, h1a, True)

    @pl.when(c == 1)
    def _():
        combine(ps1, h1b, False)


def _sc_layer1(node, srcf, dst3d, ps0, ps1, pn0, pn1, b1):
    mesh = plsc.VectorSubcoreMesh(core_axis_name="c", subcore_axis_name="s",
                                  num_cores=NC, num_subcores=NS)
    kern = pl.kernel(
        _sc1_body,
        out_type=[jax.ShapeDtypeStruct((N_NODES, HALF), f32),
                  jax.ShapeDtypeStruct((N_NODES, HALF), f32),
                  jax.ShapeDtypeStruct((N_NODES,), f32),
                  jax.ShapeDtypeStruct((N_NODES, HALF), f32),
                  jax.ShapeDtypeStruct((N_NODES, HALF), f32)],
        mesh=mesh,
        scratch_types=[
            pltpu.VMEM_SHARED((N_NODES, HALF), f32),
            pltpu.VMEM_SHARED((DEG_PAD,), f32),
            pltpu.VMEM((EPT,), i32),
            pltpu.VMEM((KCH, K), i32),
            pltpu.VMEM((K, HALF), f32),
            pltpu.VMEM((K, HALF), f32),
            pltpu.VMEM((K,), f32),
            pltpu.VMEM((HALF,), f32),
            pltpu.VMEM((K,), i32),
            pltpu.VMEM((K,), f32),
            pltpu.SemaphoreType.DMA,
            pltpu.SemaphoreType.DMA,
            pltpu.SemaphoreType.DMA,
        ],
    )
    h1a, h1b, rdeg, _ta, _tb = kern(node, srcf, dst3d, ps0, ps1, pn0, pn1, b1)
    return h1a, h1b, rdeg


def _sc2_body(src_hbm, dst_hbm, s2a, s2b, g2a, g2b, rdeg_hbm, b2_hbm,
              h2_out,
              acc, isrc, idst, rows, rows2, degblk, b2buf, sem_a, sem_b):
    c = lax.axis_index("c")
    s = lax.axis_index("s")

    _zero_vmem_2d(rows, K)
    def zblk(q, carry):
        b = s * BPT + q

        @pl.when(b < NBLK)
        def _():
            pltpu.sync_copy(rows, acc.at[pl.ds(b * K, K)])
        return carry
    lax.fori_loop(0, BPT, zblk, None)
    plsc.subcore_barrier()

    pltpu.sync_copy(src_hbm.at[pl.ds(s * EPT, EPT)], isrc)
    pltpu.sync_copy(dst_hbm.at[s], idst)

    @pl.when(c == 0)
    def _():
        _agg_pipe(g2a, isrc, idst, rows, rows2, acc, sem_a, sem_b)

    @pl.when(c == 1)
    def _():
        _agg_pipe(g2b, isrc, idst, rows, rows2, acc, sem_a, sem_b)

    plsc.subcore_barrier()

    pltpu.sync_copy(b2_hbm.at[pl.ds(c * HALF, HALF)], b2buf)

    def combine(s_ref):
        def blk(j, carry):
            b = s * BPT + j

            @pl.when(b < NBLK)
            def _():
                base = b * K
                pltpu.sync_copy(s_ref.at[pl.ds(base, K)], rows)
                pltpu.sync_copy(acc.at[pl.ds(base, K)], rows2)
                pltpu.sync_copy(rdeg_hbm.at[pl.ds(base, K)], degblk)

                def tfn(t, carry2):
                    rdvec = degblk[pl.ds(t * LANES, LANES)]
                    for l in range(LANES):
                        i = t * LANES + l
                        rd = lax.broadcast_in_dim(
                            lax.slice(rdvec, (l,), (l + 1,)), (LANES,), (0,))
                        for jj in range(HALF // LANES):
                            sl = pl.ds(jj * LANES, LANES)
                            rows[i, sl] = (rows[i, sl] + rows2[i, sl] * rd
                                           + b2buf[sl])
                    return carry2
                lax.fori_loop(0, K // LANES, tfn, None)
                pltpu.sync_copy(
                    rows, h2_out.at[pl.ds(base, K), pl.ds(c * HALF, HALF)])
            return carry
        lax.fori_loop(0, BPT, blk, None)

    @pl.when(c == 0)
    def _():
        combine(s2a)

    @pl.when(c == 1)
    def _():
        combine(s2b)


def _sc_layer2(srcf, dst3d, s2a, s2b, g2a, g2b, rdeg, b2):
    mesh = plsc.VectorSubcoreMesh(core_axis_name="c", subcore_axis_name="s",
                                  num_cores=NC, num_subcores=NS)
    kern = pl.kernel(
        _sc2_body,
        out_type=jax.ShapeDtypeStruct((N_NODES, D_H), f32),
        mesh=mesh,
        scratch_types=[
            pltpu.VMEM_SHARED((N_NODES, HALF), f32),
            pltpu.VMEM((EPT,), i32),
            pltpu.VMEM((KCH, K), i32),
            pltpu.VMEM((K, HALF), f32),
            pltpu.VMEM((K, HALF), f32),
            pltpu.VMEM((K,), f32),
            pltpu.VMEM((HALF,), f32),
            pltpu.SemaphoreType.DMA,
            pltpu.SemaphoreType.DMA,
        ],
    )
    return kern(srcf, dst3d, s2a, s2b, g2a, g2b, rdeg, b2)


def kernel(node, edge_index, emb, W_self1, W_neigh1, b1, W_self2, W_neigh2, b2):
    srcf = edge_index[0]
    dst3d = edge_index[1].reshape(NS, KCH, K)
    ps0, ps1, pn0, pn1 = _project_l1(emb, W_self1, W_neigh1)
    h1a, h1b, rdeg = _sc_layer1(node, srcf, dst3d, ps0, ps1, pn0, pn1, b1)
    s2a, s2b, g2a, g2b = _project_l2(h1a, h1b, W_self2, W_neigh2)
    return _sc_layer2(srcf, dst3d, s2a, s2b, g2a, g2b, rdeg, b2)
